# baseline (speedup 1.0000x reference)
import jax
import jax.numpy as jnp
from jax.experimental import pallas as pl
from jax.experimental.pallas import tpu as pltpu

_SAMPLE_N = 16384
_FEAT = 128
_NCHUNK = 8
_CHUNK = _SAMPLE_N // _NCHUNK


def _dma_body(x_hbm, o_hbm, sems):
    copies = [
        pltpu.make_async_copy(
            x_hbm.at[pl.ds(i * _CHUNK, _CHUNK), :],
            o_hbm.at[pl.ds(i * _CHUNK, _CHUNK), :],
            sems.at[i],
        )
        for i in range(_NCHUNK)
    ]
    for c in copies:
        c.start()
    for c in copies:
        c.wait()


def kernel(dataset):
    return pl.pallas_call(
        _dma_body,
        in_specs=[pl.BlockSpec(memory_space=pltpu.MemorySpace.HBM)],
        out_specs=pl.BlockSpec(memory_space=pltpu.MemorySpace.HBM),
        out_shape=jax.ShapeDtypeStruct((_SAMPLE_N, _FEAT), jnp.float32),
        scratch_shapes=[pltpu.SemaphoreType.DMA((_NCHUNK,))],
    )(dataset)


# speedup vs baseline: 34.2462x; 34.2462x over previous
import jax
import jax.numpy as jnp
from jax.experimental import pallas as pl

_SAMPLE_N = 16384
_FEAT = 128
_BLOCK = 4096


def _copy_body(x_ref, o_ref):
    o_ref[...] = x_ref[...]


def kernel(dataset):
    return pl.pallas_call(
        _copy_body,
        grid=(_SAMPLE_N // _BLOCK,),
        in_specs=[pl.BlockSpec((_BLOCK, _FEAT), lambda i: (i, 0))],
        out_specs=pl.BlockSpec((_BLOCK, _FEAT), lambda i: (i, 0)),
        out_shape=jax.ShapeDtypeStruct((_SAMPLE_N, _FEAT), jnp.float32),
    )(dataset)


# speedup vs baseline: 42.9029x; 1.2528x over previous
import jax
import jax.numpy as jnp
from jax.experimental import pallas as pl

_SAMPLE_N = 16384
_FEAT = 128
_BLOCK = 8192


def _copy_body(x_ref, o_ref):
    o_ref[...] = x_ref[...]


def kernel(dataset):
    return pl.pallas_call(
        _copy_body,
        grid=(_SAMPLE_N // _BLOCK,),
        in_specs=[pl.BlockSpec((_BLOCK, _FEAT), lambda i: (i, 0))],
        out_specs=pl.BlockSpec((_BLOCK, _FEAT), lambda i: (i, 0)),
        out_shape=jax.ShapeDtypeStruct((_SAMPLE_N, _FEAT), jnp.float32),
    )(dataset)
